# baseline (speedup 1.0000x reference)
import functools

import jax
import jax.numpy as jnp
from jax import lax
from jax.experimental import pallas as pl
from jax.experimental.pallas import tpu as pltpu
from jax.experimental.pallas import tpu_sc as plsc

NUM_EMBEDDINGS = 1000000
EMBEDDING_DIM = 32
BATCH = 16384

_info = plsc.get_sparse_core_info()
_NC, _NS, _NL = _info.num_cores, _info.num_subcores, _info.num_lanes
_NW = _NC * _NS
_B_PER_W = BATCH // _NW
_NBUF = 16
_WIN = 128
_CLAMP = 7811 * 128
_TAIL0 = 7812 * 128
_NTAIL = NUM_EMBEDDINGS - _TAIL0

_mesh = plsc.VectorSubcoreMesh(core_axis_name="c", subcore_axis_name="s")


@functools.partial(
    pl.kernel,
    mesh=_mesh,
    out_type=jax.ShapeDtypeStruct((EMBEDDING_DIM, BATCH), jnp.float32),
    scratch_types=[
        pltpu.VMEM((_B_PER_W,), jnp.int32),
        pltpu.VMEM((EMBEDDING_DIM, _B_PER_W), jnp.float32),
        pltpu.VMEM((_NTAIL, EMBEDDING_DIM), jnp.float32),
    ]
    + [pltpu.VMEM((EMBEDDING_DIM, _WIN), jnp.float32) for _ in range(_NBUF)]
    + [pltpu.SemaphoreType.DMA for _ in range(_NBUF)],
    compiler_params=pltpu.CompilerParams(needs_layout_passes=False),
)
def _sc_gather(
    tp_hbm, idx_hbm, tail_hbm, out_hbm, idx_v, out_v, tail_v, *bufs_sems
):
    bufs = bufs_sems[:_NBUF]
    sems = bufs_sems[_NBUF:]
    wid = lax.axis_index("s") * _NC + lax.axis_index("c")
    base = wid * _B_PER_W
    pltpu.sync_copy(idx_hbm.at[pl.ds(base, _B_PER_W)], idx_v)
    pltpu.sync_copy(tail_hbm, tail_v)
    lane = lax.iota(jnp.int32, _NL)

    def xat(i):
        c0 = lax.bitwise_and(i, jnp.int32(-_NL))
        chunk = idx_v[pl.ds(c0, _NL)]
        sel = lane == (i - c0)
        return jnp.max(jnp.where(sel, chunk, jnp.int32(0)))

    def fetch(i, b):
        x = xat(i)
        s = lax.min(lax.bitwise_and(x, jnp.int32(-128)), jnp.int32(_CLAMP))
        pltpu.async_copy(
            tp_hbm.at[:, pl.ds(pl.multiple_of(s, 128), _WIN)], bufs[b], sems[b]
        )

    for b in range(_NBUF):
        fetch(jnp.int32(b), b)

    def outer(g, carry):
        for b in range(_NBUF):
            i = g * _NBUF + b
            pltpu.make_async_copy(
                tp_hbm.at[:, pl.ds(0, _WIN)], bufs[b], sems[b]
            ).wait()
            x = xat(i)
            s = lax.min(lax.bitwise_and(x, jnp.int32(-128)), jnp.int32(_CLAMP))
            m = lax.min(x - s, jnp.int32(_WIN - 1))
            m_splat = jnp.full((_NL,), m, jnp.int32)
            lo = plsc.load_gather(bufs[b], [lane, m_splat])
            hi = plsc.load_gather(bufs[b], [lane + _NL, m_splat])
            rt = lax.max(x - jnp.int32(_TAIL0), jnp.int32(0))
            rt_splat = jnp.full((_NL,), rt, jnp.int32)
            tlo = plsc.load_gather(tail_v, [rt_splat, lane])
            thi = plsc.load_gather(tail_v, [rt_splat, lane + _NL])
            use_tail = jnp.full((_NL,), x >= _TAIL0, jnp.bool_)
            lo = jnp.where(use_tail, tlo, lo)
            hi = jnp.where(use_tail, thi, hi)
            i_splat = jnp.full((_NL,), i, jnp.int32)
            plsc.store_scatter(out_v, [lane, i_splat], lo)
            plsc.store_scatter(out_v, [lane + _NL, i_splat], hi)
            fetch(lax.min(i + _NBUF, jnp.int32(_B_PER_W - 1)), b)
        return carry

    lax.fori_loop(0, _B_PER_W // _NBUF, outer, jnp.int32(0))
    for b in range(_NBUF):
        pltpu.make_async_copy(
            tp_hbm.at[:, pl.ds(0, _WIN)], bufs[b], sems[b]
        ).wait()
    pltpu.sync_copy(out_v, out_hbm.at[:, pl.ds(base, _B_PER_W)])


def kernel(x, table):
    tail = table[_TAIL0:]
    out_t = _sc_gather(table.T, x, tail)
    return out_t.T


# speedup vs baseline: 1.0560x; 1.0560x over previous
import functools

import jax
import jax.numpy as jnp
from jax import lax
from jax.experimental import pallas as pl
from jax.experimental.pallas import tpu as pltpu
from jax.experimental.pallas import tpu_sc as plsc

NUM_EMBEDDINGS = 1000000
EMBEDDING_DIM = 32
BATCH = 16384

_info = plsc.get_sparse_core_info()
_NC, _NS, _NL = _info.num_cores, _info.num_subcores, _info.num_lanes
_NW = _NC * _NS
_B_PER_W = BATCH // _NW
_NBUF = 8
_WIN = 128
_CLAMP = 7811 * 128
_TAIL0 = 7812 * 128
_NTAIL = NUM_EMBEDDINGS - _TAIL0

_mesh = plsc.VectorSubcoreMesh(core_axis_name="c", subcore_axis_name="s")


@functools.partial(
    pl.kernel,
    mesh=_mesh,
    out_type=jax.ShapeDtypeStruct((EMBEDDING_DIM, BATCH), jnp.float32),
    scratch_types=[
        pltpu.VMEM((_B_PER_W,), jnp.int32),
        pltpu.VMEM((EMBEDDING_DIM, _B_PER_W), jnp.float32),
        pltpu.VMEM((_NTAIL, EMBEDDING_DIM), jnp.float32),
    ]
    + [pltpu.VMEM((EMBEDDING_DIM, _WIN), jnp.float32) for _ in range(_NBUF)]
    + [pltpu.SemaphoreType.DMA for _ in range(_NBUF)],
    compiler_params=pltpu.CompilerParams(needs_layout_passes=False),
)
def _sc_gather(
    tp_hbm, idx_hbm, tail_hbm, out_hbm, idx_v, out_v, tail_v, *bufs_sems
):
    bufs = bufs_sems[:_NBUF]
    sems = bufs_sems[_NBUF:]
    wid = lax.axis_index("s") * _NC + lax.axis_index("c")
    base = wid * _B_PER_W
    pltpu.sync_copy(idx_hbm.at[pl.ds(base, _B_PER_W)], idx_v)
    pltpu.sync_copy(tail_hbm, tail_v)
    lane = lax.iota(jnp.int32, _NL)

    def xat(i):
        c0 = lax.bitwise_and(i, jnp.int32(-_NL))
        chunk = idx_v[pl.ds(c0, _NL)]
        sel = lane == (i - c0)
        return jnp.max(jnp.where(sel, chunk, jnp.int32(0)))

    def fetch(i, b):
        x = xat(i)
        s = lax.min(lax.bitwise_and(x, jnp.int32(-128)), jnp.int32(_CLAMP))
        s = pl.multiple_of(s, 128)
        for tr in range(4):
            pltpu.async_copy(
                tp_hbm.at[pl.ds(8 * tr, 8), pl.ds(s, _WIN)],
                bufs[b].at[pl.ds(8 * tr, 8)],
                sems[b],
            )

    for b in range(_NBUF):
        fetch(jnp.int32(b), b)

    def outer(g, carry):
        for b in range(_NBUF):
            i = g * _NBUF + b
            pltpu.make_async_copy(
                tp_hbm.at[:, pl.ds(0, _WIN)], bufs[b], sems[b]
            ).wait()
            x = xat(i)
            s = lax.min(lax.bitwise_and(x, jnp.int32(-128)), jnp.int32(_CLAMP))
            m = lax.min(x - s, jnp.int32(_WIN - 1))
            m_splat = jnp.full((_NL,), m, jnp.int32)
            lo = plsc.load_gather(bufs[b], [lane, m_splat])
            hi = plsc.load_gather(bufs[b], [lane + _NL, m_splat])
            rt = lax.max(x - jnp.int32(_TAIL0), jnp.int32(0))
            rt_splat = jnp.full((_NL,), rt, jnp.int32)
            tlo = plsc.load_gather(tail_v, [rt_splat, lane])
            thi = plsc.load_gather(tail_v, [rt_splat, lane + _NL])
            use_tail = jnp.full((_NL,), x >= _TAIL0, jnp.bool_)
            lo = jnp.where(use_tail, tlo, lo)
            hi = jnp.where(use_tail, thi, hi)
            i_splat = jnp.full((_NL,), i, jnp.int32)
            plsc.store_scatter(out_v, [lane, i_splat], lo)
            plsc.store_scatter(out_v, [lane + _NL, i_splat], hi)
            fetch(lax.min(i + _NBUF, jnp.int32(_B_PER_W - 1)), b)
        return carry

    lax.fori_loop(0, _B_PER_W // _NBUF, outer, jnp.int32(0))
    for b in range(_NBUF):
        pltpu.make_async_copy(
            tp_hbm.at[:, pl.ds(0, _WIN)], bufs[b], sems[b]
        ).wait()
    pltpu.sync_copy(out_v, out_hbm.at[:, pl.ds(base, _B_PER_W)])


def kernel(x, table):
    tail = table[_TAIL0:]
    out_t = _sc_gather(table.T, x, tail)
    return out_t.T


# speedup vs baseline: 1.0569x; 1.0009x over previous
import functools

import jax
import jax.numpy as jnp
from jax import lax
from jax.experimental import pallas as pl
from jax.experimental.pallas import tpu as pltpu
from jax.experimental.pallas import tpu_sc as plsc

NUM_EMBEDDINGS = 1000000
EMBEDDING_DIM = 32
BATCH = 16384

_info = plsc.get_sparse_core_info()
_NC, _NS, _NL = _info.num_cores, _info.num_subcores, _info.num_lanes
_NW = _NC * _NS
_B_PER_W = BATCH // _NW
_NBUF = 8
_WIN = 128
_CLAMP = 7811 * 128
_TAIL0 = 7812 * 128
_NTAIL = NUM_EMBEDDINGS - _TAIL0

_mesh = plsc.VectorSubcoreMesh(core_axis_name="c", subcore_axis_name="s")


@functools.partial(
    pl.kernel,
    mesh=_mesh,
    out_type=jax.ShapeDtypeStruct((EMBEDDING_DIM, BATCH), jnp.float32),
    scratch_types=[
        pltpu.VMEM((_B_PER_W,), jnp.int32),
        pltpu.VMEM((EMBEDDING_DIM, _B_PER_W), jnp.float32),
        pltpu.VMEM((_NTAIL * EMBEDDING_DIM,), jnp.float32),
    ]
    + [pltpu.VMEM((EMBEDDING_DIM, _WIN), jnp.float32) for _ in range(_NBUF)]
    + [pltpu.SemaphoreType.DMA for _ in range(_NBUF)],
    compiler_params=pltpu.CompilerParams(needs_layout_passes=False),
)
def _sc_gather(
    tp_hbm, idx_hbm, tail_hbm, out_hbm, idx_v, out_v, tail_v, *bufs_sems
):
    bufs = bufs_sems[:_NBUF]
    sems = bufs_sems[_NBUF:]
    wid = lax.axis_index("s") * _NC + lax.axis_index("c")
    base = wid * _B_PER_W
    pltpu.sync_copy(idx_hbm.at[pl.ds(base, _B_PER_W)], idx_v)
    pltpu.sync_copy(tail_hbm, tail_v)
    lane = lax.iota(jnp.int32, _NL)

    def xat(i):
        c0 = lax.bitwise_and(i, jnp.int32(-_NL))
        chunk = idx_v[pl.ds(c0, _NL)]
        sel = lane == (i - c0)
        return jnp.max(jnp.where(sel, chunk, jnp.int32(0)))

    def fetch(i, b):
        x = xat(i)
        s = lax.min(lax.bitwise_and(x, jnp.int32(-128)), jnp.int32(_CLAMP))
        s = pl.multiple_of(s, 128)
        for tr in range(4):
            pltpu.async_copy(
                tp_hbm.at[pl.ds(8 * tr, 8), pl.ds(s, _WIN)],
                bufs[b].at[pl.ds(8 * tr, 8)],
                sems[b],
            )

    for b in range(_NBUF):
        fetch(jnp.int32(b), b)

    def outer(g, carry):
        for b in range(_NBUF):
            i = g * _NBUF + b
            pltpu.make_async_copy(
                tp_hbm.at[:, pl.ds(0, _WIN)], bufs[b], sems[b]
            ).wait()
            x = xat(i)
            s = lax.min(lax.bitwise_and(x, jnp.int32(-128)), jnp.int32(_CLAMP))
            m = lax.min(x - s, jnp.int32(_WIN - 1))
            m_splat = jnp.full((_NL,), m, jnp.int32)
            lo = plsc.load_gather(bufs[b], [lane, m_splat])
            hi = plsc.load_gather(bufs[b], [lane + _NL, m_splat])
            rt = lax.max(x - jnp.int32(_TAIL0), jnp.int32(0))
            tsrc = jnp.full((_NL,), rt * EMBEDDING_DIM, jnp.int32) + lane
            tlo = plsc.load_gather(tail_v, [tsrc])
            thi = plsc.load_gather(tail_v, [tsrc + _NL])
            use_tail = jnp.full((_NL,), x >= _TAIL0, jnp.bool_)
            lo = jnp.where(use_tail, tlo, lo)
            hi = jnp.where(use_tail, thi, hi)
            i_splat = jnp.full((_NL,), i, jnp.int32)
            plsc.store_scatter(out_v, [lane, i_splat], lo)
            plsc.store_scatter(out_v, [lane + _NL, i_splat], hi)
            fetch(lax.min(i + _NBUF, jnp.int32(_B_PER_W - 1)), b)
        return carry

    lax.fori_loop(0, _B_PER_W // _NBUF, outer, jnp.int32(0))
    for b in range(_NBUF):
        pltpu.make_async_copy(
            tp_hbm.at[:, pl.ds(0, _WIN)], bufs[b], sems[b]
        ).wait()
    pltpu.sync_copy(out_v, out_hbm.at[:, pl.ds(base, _B_PER_W)])


def kernel(x, table):
    tail = table[_TAIL0:].reshape(_NTAIL * EMBEDDING_DIM)
    out_t = _sc_gather(table.T, x, tail)
    return out_t.T
